# baseline (speedup 1.0000x reference)
import functools

import jax
import jax.numpy as jnp
from jax import lax
from jax.experimental import pallas as pl
from jax.experimental.pallas import tpu as pltpu
from jax.experimental.pallas import tpu_sc as plsc

N = 10000
E = 160000
D = 256
DH = 128
NS = 16
CHUNK = 120
CHUNKS_PER_SUB = 84
E_PAD = NS * CHUNKS_PER_SUB * CHUNK
EDGES_PER_SUB = E_PAD // NS
AGG_ROWS = 10240
ROWS_PER_SUB = AGG_ROWS // NS
WB = 128

_sc_mesh = plsc.VectorSubcoreMesh(core_axis_name="c", subcore_axis_name="s")


@functools.partial(
    pl.kernel,
    mesh=_sc_mesh,
    out_type=jax.ShapeDtypeStruct((2 * AGG_ROWS, DH), jnp.float32),
    scratch_types=(
        [pltpu.VMEM((CHUNK,), jnp.int32)] * 12
        + [pltpu.VMEM((CHUNK, DH), jnp.float32)] * 3
        + [pltpu.SemaphoreType.DMA] * 6
        + [pltpu.SemaphoreType.DMA] * 3
        + [pltpu.SemaphoreType.DMA] * 3
        + [pltpu.VMEM_SHARED((AGG_ROWS, DH), jnp.float32)]
    ),
)
def _sc_aggregate(xr, srcb, dst1, zeros_in, agg01,
                  s00, s01, s10, s11, s20, s21,
                  d00, d01, d10, d11, d20, d21,
                  r0_, r1_, r2_,
                  i00, i01, i10, i11, i20, i21,
                  gA, gB, gC, tA, tB, tC, agg_sh):
    c = lax.axis_index("c")
    s = lax.axis_index("s")
    H = 64
    H2 = CHUNK - 64
    sidx = ((s00, s01), (s10, s11), (s20, s21))
    didx = ((d00, d01), (d10, d11), (d20, d21))
    rows = (r0_, r1_, r2_)
    isem = ((i00, i01), (i10, i11), (i20, i21))
    gsem = (gA, gB, gC)
    ssem = (tA, tB, tC)

    pltpu.sync_copy(zeros_in, r0_)
    base = s * ROWS_PER_SUB
    nz = ROWS_PER_SUB // CHUNK
    for k in range(nz):
        pltpu.sync_copy(r0_, agg_sh.at[pl.ds(base + k * CHUNK, CHUNK)])
    ztail = ROWS_PER_SUB - nz * CHUNK
    pltpu.sync_copy(r0_.at[pl.ds(0, ztail)],
                    agg_sh.at[pl.ds(base + nz * CHUNK, ztail)])

    plsc.subcore_barrier()

    ebase0 = s * EDGES_PER_SUB
    sbase0 = c * E_PAD + ebase0

    def _src_at(g):
        return srcb.at[pl.ds(sbase0 + g * CHUNK, CHUNK)]

    def _dst_at(g):
        return dst1.at[pl.ds(ebase0 + g * CHUNK, CHUNK)]

    def _fire_gather(g, r):
        j, p = r % 3, (r // 3) % 2
        sx, rw, sm = sidx[j][p], rows[j], gsem[j]
        pltpu.async_copy(xr.at[sx.at[pl.ds(0, H)]], rw.at[pl.ds(0, H)], sm)
        pltpu.async_copy(xr.at[sx.at[pl.ds(H, H2)]], rw.at[pl.ds(H, H2)], sm)

    def _wait_gather(g, r):
        j, p = r % 3, (r // 3) % 2
        sx, rw, sm = sidx[j][p], rows[j], gsem[j]
        pltpu.make_async_copy(xr.at[sx.at[pl.ds(0, H)]],
                              rw.at[pl.ds(0, H)], sm).wait()
        pltpu.make_async_copy(xr.at[sx.at[pl.ds(H, H2)]],
                              rw.at[pl.ds(H, H2)], sm).wait()

    def _fire_idx(g, r):
        j, p = r % 3, (r // 3) % 2
        pltpu.make_async_copy(_src_at(g), sidx[j][p], isem[j][p]).start()
        pltpu.make_async_copy(_dst_at(g), didx[j][p], isem[j][p]).start()

    def _wait_idx(g, r):
        j, p = r % 3, (r // 3) % 2
        pltpu.make_async_copy(_src_at(g), sidx[j][p], isem[j][p]).wait()
        pltpu.make_async_copy(_dst_at(g), didx[j][p], isem[j][p]).wait()

    def _fire_scat(g, r):
        j, p = r % 3, (r // 3) % 2
        pltpu.async_copy(rows[j], agg_sh.at[didx[j][p]], ssem[j], add=True)

    def _wait_scat(g, r):
        j, p = r % 3, (r // 3) % 2
        pltpu.make_async_copy(rows[j], agg_sh.at[didx[j][p]], ssem[j]).wait()

    for g0 in range(5):
        _fire_idx(g0, g0)
    _wait_idx(0, 0)
    _fire_gather(0, 0)
    _wait_idx(1, 1)
    _fire_gather(1, 1)

    def _emit_chunk(g, r, first=False, last=False):
        _wait_gather(g, r)
        _fire_scat(g, r)
        if not first:
            _wait_scat(g - 1, (r + 5) % 6)
        if not last:
            _fire_idx(g + 5, (r + 5) % 6)
        _wait_idx(g + 2, (r + 2) % 6)
        _fire_gather(g + 2, (r + 2) % 6)

    def _emit_body(t, first=False, last=False):
        g = 6 * t
        for u in range(6):
            if last and u >= 4:
                _wait_gather(g + u, u)
                _fire_scat(g + u, u)
                _wait_scat(g + u - 1, (u + 5) % 6)
            else:
                _emit_chunk(g + u, u, first=(first and u == 0),
                            last=(last and u >= 1))
        return

    def _body(t, carry):
        _emit_body(t)
        return carry

    _emit_body(0, first=True)
    lax.fori_loop(1, CHUNKS_PER_SUB // 6 - 1, _body, 0)
    _emit_body(CHUNKS_PER_SUB // 6 - 1, last=True)

    _wait_scat(CHUNKS_PER_SUB - 1, (CHUNKS_PER_SUB - 1) % 6)

    plsc.subcore_barrier()

    obase = c * AGG_ROWS + base
    for k in range(nz):
        pltpu.sync_copy(agg_sh.at[pl.ds(base + k * CHUNK, CHUNK)], r0_)
        pltpu.sync_copy(r0_, agg01.at[pl.ds(obase + k * CHUNK, CHUNK)])
    pltpu.sync_copy(agg_sh.at[pl.ds(base + nz * CHUNK, ztail)],
                    r0_.at[pl.ds(0, ztail)])
    pltpu.sync_copy(r0_.at[pl.ds(0, ztail)],
                    agg01.at[pl.ds(obase + nz * CHUNK, ztail)])


DEG_W = 128


@functools.partial(
    pl.kernel,
    mesh=_sc_mesh,
    out_type=jax.ShapeDtypeStruct((2 * AGG_ROWS, DEG_W), jnp.float32),
    scratch_types=[
        pltpu.VMEM((CHUNK,), jnp.int32),
        pltpu.VMEM((CHUNK,), jnp.int32),
        pltpu.VMEM((WB, DEG_W), jnp.float32),
        pltpu.SemaphoreType.DMA,
        pltpu.SemaphoreType.DMA,
        pltpu.VMEM_SHARED((AGG_ROWS, DEG_W), jnp.float32),
    ],
)
def _sc_degree(dst1, zeros16_in, ones_in, deg01, dA, dB, ones_v, iA, iB,
               deg_sh):
    c = lax.axis_index("c")
    s = lax.axis_index("s")

    base = s * ROWS_PER_SUB
    pltpu.sync_copy(zeros16_in, ones_v)
    for k in range(4):
        pltpu.sync_copy(ones_v, deg_sh.at[pl.ds(base + k * WB, WB)])
    tail = ROWS_PER_SUB - 4 * WB
    pltpu.sync_copy(ones_v.at[pl.ds(0, tail)],
                    deg_sh.at[pl.ds(base + 4 * WB, tail)])
    pltpu.sync_copy(ones_in, ones_v)
    plsc.subcore_barrier()

    nch = CHUNKS_PER_SUB // 2
    cbase = c * (E_PAD // 2)

    def _dst_at(g):
        return dst1.at[pl.ds(cbase + (s * nch + g) * CHUNK, CHUNK)]

    pltpu.sync_copy(_dst_at(0), dA)
    pltpu.make_async_copy(_dst_at(1), dB, iB).start()

    def _chunk(g, carry):
        pltpu.sync_copy(ones_v.at[pl.ds(0, CHUNK)], deg_sh.at[dA], add=True)
        pltpu.make_async_copy(_dst_at(2 * g + 1), dB, iB).wait()
        pltpu.make_async_copy(_dst_at(2 * g + 2), dA, iA).start()
        pltpu.sync_copy(ones_v.at[pl.ds(0, CHUNK)], deg_sh.at[dB], add=True)
        pltpu.make_async_copy(_dst_at(2 * g + 2), dA, iA).wait()
        pltpu.make_async_copy(_dst_at(2 * g + 3), dB, iB).start()
        return carry
    lax.fori_loop(0, nch // 2 - 1, _chunk, 0)

    pltpu.sync_copy(ones_v.at[pl.ds(0, CHUNK)], deg_sh.at[dA], add=True)
    pltpu.make_async_copy(_dst_at(nch - 1), dB, iB).wait()
    pltpu.sync_copy(ones_v.at[pl.ds(0, CHUNK)], deg_sh.at[dB], add=True)

    plsc.subcore_barrier()

    obase = c * AGG_ROWS + base
    for k in range(4):
        pltpu.sync_copy(deg_sh.at[pl.ds(base + k * WB, WB)], ones_v)
        pltpu.sync_copy(ones_v, deg01.at[pl.ds(obase + k * WB, WB)])
    pltpu.sync_copy(deg_sh.at[pl.ds(base + 4 * WB, tail)],
                    ones_v.at[pl.ds(0, tail)])
    pltpu.sync_copy(ones_v.at[pl.ds(0, tail)],
                    deg01.at[pl.ds(obase + 4 * WB, tail)])


_BLK = 80


def _tc_body(x_ref, ws_ref, wn0_ref, wn1_ref, b_ref, a0_ref, a1_ref,
             d0_ref, d1_ref, out_ref):
    inv = 1.0 / jnp.maximum(d0_ref[:, 0:1] + d1_ref[:, 0:1], 1.0)
    dn = (((1,), (1,)), ((), ()))
    acc = lax.dot_general(x_ref[...], ws_ref[...], dn,
                          preferred_element_type=jnp.float32)
    acc += lax.dot_general(a0_ref[...] * inv, wn0_ref[...], dn,
                           preferred_element_type=jnp.float32)
    acc += lax.dot_general(a1_ref[...] * inv, wn1_ref[...], dn,
                           preferred_element_type=jnp.float32)
    out_ref[...] = acc + b_ref[...][None, :]


def _tc_combine(x, W_self, Wn0, Wn1, b, agg01, deg01):
    grid = (N // _BLK,)
    off = AGG_ROWS // _BLK
    return pl.pallas_call(
        _tc_body,
        grid=grid,
        in_specs=[
            pl.BlockSpec((_BLK, D), lambda i: (i, 0)),
            pl.BlockSpec((D, D), lambda i: (0, 0)),
            pl.BlockSpec((D, DH), lambda i: (0, 0)),
            pl.BlockSpec((D, DH), lambda i: (0, 0)),
            pl.BlockSpec((D,), lambda i: (0,)),
            pl.BlockSpec((_BLK, DH), lambda i: (i, 0)),
            pl.BlockSpec((_BLK, DH), lambda i: (off + i, 0)),
            pl.BlockSpec((_BLK, DEG_W), lambda i: (i, 0)),
            pl.BlockSpec((_BLK, DEG_W), lambda i: (off + i, 0)),
        ],
        out_specs=pl.BlockSpec((_BLK, D), lambda i: (i, 0)),
        out_shape=jax.ShapeDtypeStruct((N, D), jnp.float32),
    )(x, W_self, Wn0, Wn1, b, agg01, agg01, deg01, deg01)


def kernel(x, edge_index, W_self, W_neigh, b):
    src = edge_index[0]
    dst = edge_index[1]
    pad = E_PAD - E
    src1 = jnp.concatenate([src, jnp.zeros((pad,), jnp.int32)])
    dst1 = jnp.concatenate([dst, jnp.full((pad,), N, jnp.int32)])
    xr = x.reshape(2 * N, DH)
    srcb = jnp.concatenate([2 * src1, 2 * src1 + 1])
    zeros_in = jnp.zeros((CHUNK, DH), jnp.float32)
    ones_in = jnp.ones((WB, DEG_W), jnp.float32)
    zeros16_in = jnp.zeros((WB, DEG_W), jnp.float32)

    agg01 = _sc_aggregate(xr, srcb, dst1, zeros_in)
    deg01 = _sc_degree(dst1, zeros16_in, ones_in)

    Wn0 = W_neigh[:, :DH]
    Wn1 = W_neigh[:, DH:]
    return _tc_combine(x, W_self, Wn0, Wn1, b, agg01, deg01)


# speedup vs baseline: 1.2131x; 1.2131x over previous
import functools

import jax
import jax.numpy as jnp
from jax import lax
from jax.experimental import pallas as pl
from jax.experimental.pallas import tpu as pltpu
from jax.experimental.pallas import tpu_sc as plsc

N = 10000
E = 160000
D = 256
DH = 128
NS = 16
CHUNK = 120
CHUNKS_PER_SUB = 84
E_PAD = NS * CHUNKS_PER_SUB * CHUNK
EDGES_PER_SUB = E_PAD // NS
AGG_ROWS = 10112
ROWS_PER_SUB = AGG_ROWS // NS
WB = 128

_sc_mesh = plsc.VectorSubcoreMesh(core_axis_name="c", subcore_axis_name="s")


@functools.partial(
    pl.kernel,
    mesh=_sc_mesh,
    out_type=jax.ShapeDtypeStruct((2 * AGG_ROWS, DH), jnp.float32),
    scratch_types=(
        [pltpu.VMEM((CHUNK,), jnp.int32)] * 12
        + [pltpu.VMEM((CHUNK, DH), jnp.float32)] * 3
        + [pltpu.SemaphoreType.DMA] * 6
        + [pltpu.SemaphoreType.DMA] * 3
        + [pltpu.SemaphoreType.DMA] * 3
        + [pltpu.VMEM_SHARED((AGG_ROWS, DH), jnp.float32)]
    ),
)
def _sc_aggregate(xr, srcb, dst1, zeros_in, agg01,
                  s00, s01, s10, s11, s20, s21,
                  d00, d01, d10, d11, d20, d21,
                  r0_, r1_, r2_,
                  i00, i01, i10, i11, i20, i21,
                  gA, gB, gC, tA, tB, tC, agg_sh):
    c = lax.axis_index("c")
    s = lax.axis_index("s")
    H = 64
    H2 = CHUNK - 64
    sidx = ((s00, s01), (s10, s11), (s20, s21))
    didx = ((d00, d01), (d10, d11), (d20, d21))
    rows = (r0_, r1_, r2_)
    isem = ((i00, i01), (i10, i11), (i20, i21))
    gsem = (gA, gB, gC)
    ssem = (tA, tB, tC)

    pltpu.sync_copy(zeros_in, r0_)
    base = s * ROWS_PER_SUB
    nz = ROWS_PER_SUB // CHUNK
    for k in range(nz):
        pltpu.sync_copy(r0_, agg_sh.at[pl.ds(base + k * CHUNK, CHUNK)])
    ztail = ROWS_PER_SUB - nz * CHUNK
    pltpu.sync_copy(r0_.at[pl.ds(0, ztail)],
                    agg_sh.at[pl.ds(base + nz * CHUNK, ztail)])

    plsc.subcore_barrier()

    ebase0 = s * EDGES_PER_SUB
    sbase0 = c * E_PAD + ebase0

    def _src_at(g):
        return srcb.at[pl.ds(sbase0 + g * CHUNK, CHUNK)]

    def _dst_at(g):
        return dst1.at[pl.ds(ebase0 + g * CHUNK, CHUNK)]

    def _fire_gather(g, r):
        j, p = r % 3, (r // 3) % 2
        sx, rw, sm = sidx[j][p], rows[j], gsem[j]
        pltpu.async_copy(xr.at[sx.at[pl.ds(0, H)]], rw.at[pl.ds(0, H)], sm)
        pltpu.async_copy(xr.at[sx.at[pl.ds(H, H2)]], rw.at[pl.ds(H, H2)], sm)

    def _wait_gather(g, r):
        j, p = r % 3, (r // 3) % 2
        sx, rw, sm = sidx[j][p], rows[j], gsem[j]
        pltpu.make_async_copy(xr.at[sx.at[pl.ds(0, H)]],
                              rw.at[pl.ds(0, H)], sm).wait()
        pltpu.make_async_copy(xr.at[sx.at[pl.ds(H, H2)]],
                              rw.at[pl.ds(H, H2)], sm).wait()

    def _fire_idx(g, r):
        j, p = r % 3, (r // 3) % 2
        pltpu.make_async_copy(_src_at(g), sidx[j][p], isem[j][p]).start()
        pltpu.make_async_copy(_dst_at(g), didx[j][p], isem[j][p]).start()

    def _wait_idx(g, r):
        j, p = r % 3, (r // 3) % 2
        pltpu.make_async_copy(_src_at(g), sidx[j][p], isem[j][p]).wait()
        pltpu.make_async_copy(_dst_at(g), didx[j][p], isem[j][p]).wait()

    def _fire_scat(g, r):
        j, p = r % 3, (r // 3) % 2
        pltpu.async_copy(rows[j], agg_sh.at[didx[j][p]], ssem[j], add=True)

    def _wait_scat(g, r):
        j, p = r % 3, (r // 3) % 2
        pltpu.make_async_copy(rows[j], agg_sh.at[didx[j][p]], ssem[j]).wait()

    for g0 in range(5):
        _fire_idx(g0, g0)
    _wait_idx(0, 0)
    _fire_gather(0, 0)
    _wait_idx(1, 1)
    _fire_gather(1, 1)

    def _emit_chunk(g, r, first=False, last=False):
        _wait_gather(g, r)
        _fire_scat(g, r)
        if not first:
            _wait_scat(g - 1, (r + 5) % 6)
        if not last:
            _fire_idx(g + 5, (r + 5) % 6)
        _wait_idx(g + 2, (r + 2) % 6)
        _fire_gather(g + 2, (r + 2) % 6)

    def _emit_body(t, first=False, last=False):
        g = 6 * t
        for u in range(6):
            if last and u >= 4:
                _wait_gather(g + u, u)
                _fire_scat(g + u, u)
                _wait_scat(g + u - 1, (u + 5) % 6)
            else:
                _emit_chunk(g + u, u, first=(first and u == 0),
                            last=(last and u >= 1))
        return

    def _body(t, carry):
        _emit_body(t)
        return carry

    _emit_body(0, first=True)
    lax.fori_loop(1, CHUNKS_PER_SUB // 6 - 1, _body, 0)
    _emit_body(CHUNKS_PER_SUB // 6 - 1, last=True)

    _wait_scat(CHUNKS_PER_SUB - 1, (CHUNKS_PER_SUB - 1) % 6)

    plsc.subcore_barrier()

    obase = c * AGG_ROWS + base
    for k in range(nz):
        pltpu.sync_copy(agg_sh.at[pl.ds(base + k * CHUNK, CHUNK)], r0_)
        pltpu.sync_copy(r0_, agg01.at[pl.ds(obase + k * CHUNK, CHUNK)])
    pltpu.sync_copy(agg_sh.at[pl.ds(base + nz * CHUNK, ztail)],
                    r0_.at[pl.ds(0, ztail)])
    pltpu.sync_copy(r0_.at[pl.ds(0, ztail)],
                    agg01.at[pl.ds(obase + nz * CHUNK, ztail)])


DEG_W = 128


@functools.partial(
    pl.kernel,
    mesh=_sc_mesh,
    out_type=jax.ShapeDtypeStruct((2 * AGG_ROWS, DEG_W), jnp.float32),
    scratch_types=[
        pltpu.VMEM((CHUNK,), jnp.int32),
        pltpu.VMEM((CHUNK,), jnp.int32),
        pltpu.VMEM((WB, DEG_W), jnp.float32),
        pltpu.SemaphoreType.DMA,
        pltpu.SemaphoreType.DMA,
        pltpu.VMEM_SHARED((AGG_ROWS, DEG_W), jnp.float32),
    ],
)
def _sc_degree(dst1, zeros16_in, ones_in, deg01, dA, dB, ones_v, iA, iB,
               deg_sh):
    c = lax.axis_index("c")
    s = lax.axis_index("s")

    base = s * ROWS_PER_SUB
    pltpu.sync_copy(zeros16_in, ones_v)
    for k in range(4):
        pltpu.sync_copy(ones_v, deg_sh.at[pl.ds(base + k * WB, WB)])
    tail = ROWS_PER_SUB - 4 * WB
    pltpu.sync_copy(ones_v.at[pl.ds(0, tail)],
                    deg_sh.at[pl.ds(base + 4 * WB, tail)])
    pltpu.sync_copy(ones_in, ones_v)
    plsc.subcore_barrier()

    nch = CHUNKS_PER_SUB // 2
    cbase = c * (E_PAD // 2)

    def _dst_at(g):
        return dst1.at[pl.ds(cbase + (s * nch + g) * CHUNK, CHUNK)]

    pltpu.sync_copy(_dst_at(0), dA)
    pltpu.make_async_copy(_dst_at(1), dB, iB).start()

    def _chunk(g, carry):
        pltpu.sync_copy(ones_v.at[pl.ds(0, CHUNK)], deg_sh.at[dA], add=True)
        pltpu.make_async_copy(_dst_at(2 * g + 1), dB, iB).wait()
        pltpu.make_async_copy(_dst_at(2 * g + 2), dA, iA).start()
        pltpu.sync_copy(ones_v.at[pl.ds(0, CHUNK)], deg_sh.at[dB], add=True)
        pltpu.make_async_copy(_dst_at(2 * g + 2), dA, iA).wait()
        pltpu.make_async_copy(_dst_at(2 * g + 3), dB, iB).start()
        return carry
    lax.fori_loop(0, nch // 2 - 1, _chunk, 0)

    pltpu.sync_copy(ones_v.at[pl.ds(0, CHUNK)], deg_sh.at[dA], add=True)
    pltpu.make_async_copy(_dst_at(nch - 1), dB, iB).wait()
    pltpu.sync_copy(ones_v.at[pl.ds(0, CHUNK)], deg_sh.at[dB], add=True)

    plsc.subcore_barrier()

    obase = c * AGG_ROWS + base
    for k in range(4):
        pltpu.sync_copy(deg_sh.at[pl.ds(base + k * WB, WB)], ones_v)
        pltpu.sync_copy(ones_v, deg01.at[pl.ds(obase + k * WB, WB)])
    pltpu.sync_copy(deg_sh.at[pl.ds(base + 4 * WB, tail)],
                    ones_v.at[pl.ds(0, tail)])
    pltpu.sync_copy(ones_v.at[pl.ds(0, tail)],
                    deg01.at[pl.ds(obase + 4 * WB, tail)])


_BLK = 1000


def _tc_body(x_ref, ws_ref, wn0_ref, wn1_ref, b_ref, a0_ref, a1_ref,
             d0_ref, d1_ref, out_ref):
    inv = 1.0 / jnp.maximum(d0_ref[...] + d1_ref[...], 1.0)
    dn = (((1,), (1,)), ((), ()))
    acc = lax.dot_general(x_ref[...], ws_ref[...], dn,
                          preferred_element_type=jnp.float32)
    acc += lax.dot_general(a0_ref[...] * inv, wn0_ref[...], dn,
                           preferred_element_type=jnp.float32)
    acc += lax.dot_general(a1_ref[...] * inv, wn1_ref[...], dn,
                           preferred_element_type=jnp.float32)
    out_ref[...] = acc + b_ref[...][None, :]


def _tc_combine(x, W_self, Wn0, Wn1, b, agg0, agg1, d0, d1):
    grid = (N // _BLK,)
    return pl.pallas_call(
        _tc_body,
        grid=grid,
        in_specs=[
            pl.BlockSpec((_BLK, D), lambda i: (i, 0)),
            pl.BlockSpec((D, D), lambda i: (0, 0)),
            pl.BlockSpec((D, DH), lambda i: (0, 0)),
            pl.BlockSpec((D, DH), lambda i: (0, 0)),
            pl.BlockSpec((D,), lambda i: (0,)),
            pl.BlockSpec((_BLK, DH), lambda i: (i, 0)),
            pl.BlockSpec((_BLK, DH), lambda i: (i, 0)),
            pl.BlockSpec((_BLK, 1), lambda i: (i, 0)),
            pl.BlockSpec((_BLK, 1), lambda i: (i, 0)),
        ],
        out_specs=pl.BlockSpec((_BLK, D), lambda i: (i, 0)),
        out_shape=jax.ShapeDtypeStruct((N, D), jnp.float32),
    )(x, W_self, Wn0, Wn1, b, agg0, agg1, d0, d1)


def kernel(x, edge_index, W_self, W_neigh, b):
    src = edge_index[0]
    dst = edge_index[1]
    pad = E_PAD - E
    src1 = jnp.concatenate([src, jnp.zeros((pad,), jnp.int32)])
    dst1 = jnp.concatenate([dst, jnp.full((pad,), N, jnp.int32)])
    xr = x.reshape(2 * N, DH)
    srcb = jnp.concatenate([2 * src1, 2 * src1 + 1])
    zeros_in = jnp.zeros((CHUNK, DH), jnp.float32)
    ones_in = jnp.ones((WB, DEG_W), jnp.float32)
    zeros16_in = jnp.zeros((WB, DEG_W), jnp.float32)

    agg01 = _sc_aggregate(xr, srcb, dst1, zeros_in)
    deg01 = _sc_degree(dst1, zeros16_in, ones_in)
    agg0 = agg01[:N]
    agg1 = agg01[AGG_ROWS:AGG_ROWS + N]
    d0 = deg01[:N, :1]
    d1 = deg01[AGG_ROWS:AGG_ROWS + N, :1]

    Wn0 = W_neigh[:, :DH]
    Wn1 = W_neigh[:, DH:]
    return _tc_combine(x, W_self, Wn0, Wn1, b, agg0, agg1, d0, d1)
